# baseline (speedup 1.0000x reference)
import functools
import math

import jax
import jax.numpy as jnp
from jax import lax
from jax.experimental import pallas as pl
from jax.experimental.pallas import tpu as pltpu
from jax.experimental.pallas import tpu_sc as plsc

D_EMBED = 64
SCALE = math.sqrt(D_EMBED)
L = 16
KR = 4
NBUF = 2


def _build_sc_kernel(num_rows_x, num_cols_x):
    try:
        info = plsc.get_sparse_core_info()
        nc, ns = info.num_cores, info.num_subcores
    except Exception:
        nc, ns = 2, 16
    nw = nc * ns
    assert num_rows_x % (nw * KR) == 0
    rows_w = num_rows_x // nw
    per_w = rows_w * num_cols_x
    ck = KR * num_cols_x
    nchunk = rows_w // KR
    assert nchunk % NBUF == 0 and nchunk >= 2 * NBUF
    assert ck % 8 == 0 and per_w % L == 0

    shift = 21
    magic = -(-(1 << shift) // num_cols_x)
    assert (per_w - 1) * magic < 2**31
    assert all((q * magic) >> shift == q // num_cols_x
               for q in range(0, per_w, 997)) and (
        (per_w - 1) * magic) >> shift == (per_w - 1) // num_cols_x

    mesh = plsc.VectorSubcoreMesh(core_axis_name="c", subcore_axis_name="s")

    @functools.partial(
        pl.kernel,
        mesh=mesh,
        compiler_params=pltpu.CompilerParams(
            use_tc_tiling_on_sc=False, needs_layout_passes=False),
        out_type=jax.ShapeDtypeStruct(
            (num_rows_x, num_cols_x, D_EMBED), jnp.float32),
        scratch_types=(
            [pltpu.VMEM((rows_w, num_cols_x), jnp.int32),
             pltpu.VMEM((per_w,), jnp.int32)]
            + [pltpu.VMEM((ck, D_EMBED), jnp.float32) for _ in range(NBUF)]
            + [pltpu.VMEM((KR, num_cols_x, D_EMBED), jnp.float32)
               for _ in range(NBUF)]
            + [pltpu.SemaphoreType.DMA for _ in range(2 * NBUF)]
        ),
    )
    def emb(x_hbm, table_hbm, out_hbm, xstage, xidx, *bufs_and_sems):
        gbuf = bufs_and_sems[0:NBUF]
        sbuf = bufs_and_sems[NBUF:2 * NBUF]
        gsem = bufs_and_sems[2 * NBUF:3 * NBUF]
        ssem = bufs_and_sems[3 * NBUF:4 * NBUF]

        wid = lax.axis_index("s") * nc + lax.axis_index("c")
        row0 = wid * rows_w

        pltpu.sync_copy(x_hbm.at[pl.ds(row0, rows_w)], xstage)

        lane_iota = jax.lax.iota(jnp.int32, L)

        @plsc.parallel_loop(0, per_w // L, unroll=4)
        def _(t):
            q = lane_iota + t * L
            r = jax.lax.shift_right_logical(q * magic, shift)
            c = q - r * num_cols_x
            xidx[pl.ds(t * L, L)] = plsc.load_gather(xstage, [r, c])

        def start_gather(j, b):
            pltpu.async_copy(
                table_hbm.at[xidx.at[pl.ds(j * ck, ck)]], gbuf[b], gsem[b])

        def wait_gather(b):
            pltpu.make_async_copy(
                table_hbm.at[xidx.at[pl.ds(0, ck)]], gbuf[b], gsem[b]).wait()

        def start_scatter(j, b):
            pltpu.async_copy(
                sbuf[b], out_hbm.at[pl.ds(row0 + j * KR, KR)], ssem[b])

        def wait_scatter(b):
            pltpu.make_async_copy(
                sbuf[b], out_hbm.at[pl.ds(row0, KR)], ssem[b]).wait()

        def scale(b):
            gb, sb = gbuf[b], sbuf[b]
            for a in range(KR):

                @plsc.parallel_loop(0, num_cols_x, unroll=2)
                def _(r):
                    for c4 in range(D_EMBED // L):
                        sl = pl.ds(c4 * L, L)
                        sb[a, r, sl] = gb[a * num_cols_x + r, sl] * SCALE

        for b in range(NBUF):
            start_gather(b, b)

        for b in range(NBUF):
            wait_gather(b)
            scale(b)
            start_scatter(b, b)
            start_gather(b + NBUF, b)

        @pl.loop(NBUF, nchunk - NBUF, step=NBUF)
        def _(g):
            for b in range(NBUF):
                j = g + b
                wait_gather(b)
                wait_scatter(b)
                scale(b)
                start_scatter(j, b)
                start_gather(j + NBUF, b)

        for b in range(NBUF):
            j = nchunk - NBUF + b
            wait_gather(b)
            wait_scatter(b)
            scale(b)
            start_scatter(j, b)

        for b in range(NBUF):
            wait_scatter(b)

    return emb


def kernel(x, table):
    rows, cols = x.shape
    emb = _build_sc_kernel(rows, cols)
    return emb(x, table)
